# baseline (speedup 1.0000x reference)
import functools

import jax
import jax.numpy as jnp
from jax import lax
from jax.experimental import pallas as pl
from jax.experimental.pallas import tpu as pltpu
from jax.experimental.pallas import tpu_sc as plsc

_EPS = 1e-7
_L = 16
_NC = 2
_NS = 16
_NW = _NC * _NS
_CH = 128


@functools.lru_cache(maxsize=None)
def _build(n_rows, seq_len):
    assert n_rows % (_NW * _CH) == 0
    rows_per_w = n_rows // _NW
    n_ch = rows_per_w // _CH
    grp_per_ch = _CH // _L
    n_win = seq_len // _L
    tail = seq_len - n_win * _L
    mesh = plsc.VectorSubcoreMesh(core_axis_name="c", subcore_axis_name="s")

    @functools.partial(
        pl.kernel,
        out_type=(
            jax.ShapeDtypeStruct((n_rows,), jnp.float32),
            jax.ShapeDtypeStruct((n_rows,), jnp.float32),
        ),
        mesh=mesh,
        compiler_params=pltpu.CompilerParams(
            needs_layout_passes=False),
        scratch_types=[
            pltpu.VMEM((_CH, seq_len), jnp.float32),
            pltpu.VMEM((_CH, seq_len), jnp.float32),
            pltpu.VMEM((_CH, 2), jnp.int32),
            pltpu.VMEM((_CH, 2), jnp.int32),
            pltpu.VMEM((rows_per_w + _L,), jnp.float32),
            pltpu.VMEM((rows_per_w + _L,), jnp.float32),
            pltpu.SemaphoreType.DMA,
            pltpu.SemaphoreType.DMA,
        ],
    )
    def sc_kernel(bid_hbm, x_hbm, surv_hbm, rate_hbm,
                  xb0, xb1, bb0, bb1, sv, rv, sem0, sem1):
        wid = lax.axis_index("s") * _NC + lax.axis_index("c")
        base = wid * rows_per_w
        xbufs = (xb0, xb1)
        bbufs = (bb0, bb1)
        sems = (sem0, sem1)

        lane = lax.iota(jnp.int32, _L)
        zero_i = jnp.zeros((_L,), jnp.int32)
        one_i = jnp.full((_L,), 1, jnp.int32)
        ones_f = jnp.ones((_L,), jnp.float32)
        lane0 = lane == 0
        perms = [lane ^ 8, lane ^ 4, lane ^ 2, lane ^ 1]
        cols = [jnp.full((_L,), k * _L, jnp.int32) + lane for k in range(n_win)]
        if tail:
            cols.append(jnp.full((_L,), seq_len - _L, jnp.int32) + lane)
            tail_new = lane >= (_L - tail)

        def start(c):
            r0 = base + c * _CH
            hx = pltpu.async_copy(
                x_hbm.at[pl.ds(r0, _CH), :], xbufs[c % 2], sems[c % 2])
            hb = pltpu.async_copy(
                bid_hbm.at[pl.ds(r0, _CH), :], bbufs[c % 2], sems[c % 2])
            return hx, hb

        handles = start(0)
        for c in range(n_ch):
            nxt = start(c + 1) if c + 1 < n_ch else None
            handles[0].wait()
            handles[1].wait()
            xv = xbufs[c % 2]
            bv = bbufs[c % 2]

            def grp(g, carry):
                rows16 = g * _L + lane
                mp_vec = plsc.load_gather(bv, [rows16, zero_i])
                bid_vec = plsc.load_gather(bv, [rows16, one_i])
                xmp_vec = plsc.load_gather(xv, [rows16, mp_vec])

                @plsc.parallel_loop(0, _L, step=1, unroll=4)
                def row(jr):
                    r = g * _L + jr
                    rsp = jnp.full((_L,), jr, jnp.int32)
                    bid_r = jnp.take(bid_vec, rsp, axis=0)
                    mp_r = jnp.take(mp_vec, rsp, axis=0)

                    acc = [ones_f, ones_f, ones_f, ones_f]
                    for k in range(n_win):
                        v = xv[r, pl.ds(k * _L, _L)]
                        ck = cols[k]
                        e = k & 1
                        acc[e] = jnp.where(ck < bid_r, acc[e] * v, acc[e])
                        acc[2 + e] = jnp.where(
                            ck < mp_r, acc[2 + e] * v, acc[2 + e])
                    if tail:
                        v = xv[r, pl.ds(seq_len - _L, _L)]
                        ck = cols[n_win]
                        e = n_win & 1
                        m_s = tail_new & (ck < bid_r)
                        m_2 = tail_new & (ck < mp_r)
                        acc[e] = jnp.where(m_s, acc[e] * v, acc[e])
                        acc[2 + e] = jnp.where(m_2, acc[2 + e] * v, acc[2 + e])
                    acc_s = acc[0] * acc[1]
                    acc_2 = acc[2] * acc[3]
                    for p in perms:
                        acc_s = acc_s * jnp.take(acc_s, p, axis=0)
                        acc_2 = acc_2 * jnp.take(acc_2, p, axis=0)
                    out0 = c * _CH + r
                    plsc.store_compressed(
                        sv.at[pl.ds(out0, _L)], acc_s, mask=lane0)
                    plsc.store_compressed(
                        rv.at[pl.ds(out0, _L)], acc_2, mask=lane0)

                gout = c * _CH + g * _L
                p2 = rv[pl.ds(gout, _L)]
                rate = jnp.where(
                    mp_vec != zero_i, p2 * (1.0 - xmp_vec), jnp.float32(_EPS))
                rv[pl.ds(gout, _L)] = rate
                return carry

            lax.fori_loop(0, grp_per_ch, grp, 0)
            handles = nxt

        pltpu.sync_copy(sv.at[pl.ds(0, rows_per_w)],
                        surv_hbm.at[pl.ds(base, rows_per_w)])
        pltpu.sync_copy(rv.at[pl.ds(0, rows_per_w)],
                        rate_hbm.at[pl.ds(base, rows_per_w)])

    return sc_kernel


def kernel(bid_info, x):
    n, seq_len = x.shape
    surv, rate = _build(n, seq_len)(bid_info, x)
    return surv[:, None], rate[:, None]


# speedup vs baseline: 1.0913x; 1.0913x over previous
import functools

import jax
import jax.numpy as jnp
from jax import lax
from jax.experimental import pallas as pl
from jax.experimental.pallas import tpu as pltpu
from jax.experimental.pallas import tpu_sc as plsc

_EPS = 1e-7
_L = 16
_NC = 2
_NS = 16
_NW = _NC * _NS
_CH = 128


@functools.lru_cache(maxsize=None)
def _build(n_rows, seq_len):
    assert n_rows % (_NW * _CH) == 0
    rows_per_w = n_rows // _NW
    n_ch = rows_per_w // _CH
    grp_per_ch = _CH // _L
    n_win = seq_len // _L
    tail = seq_len - n_win * _L
    mesh = plsc.VectorSubcoreMesh(core_axis_name="c", subcore_axis_name="s")

    @functools.partial(
        pl.kernel,
        out_type=(
            jax.ShapeDtypeStruct((n_rows,), jnp.float32),
            jax.ShapeDtypeStruct((n_rows,), jnp.float32),
        ),
        mesh=mesh,
        compiler_params=pltpu.CompilerParams(
            needs_layout_passes=False),
        scratch_types=[
            pltpu.VMEM((_CH, seq_len), jnp.float32),
            pltpu.VMEM((_CH, seq_len), jnp.float32),
            pltpu.VMEM((_CH, 2), jnp.int32),
            pltpu.VMEM((_CH, 2), jnp.int32),
            pltpu.VMEM((rows_per_w + _L,), jnp.float32),
            pltpu.VMEM((rows_per_w + _L,), jnp.float32),
            pltpu.SemaphoreType.DMA,
            pltpu.SemaphoreType.DMA,
        ],
    )
    def sc_kernel(bid_hbm, x_hbm, surv_hbm, rate_hbm,
                  xb0, xb1, bb0, bb1, sv, rv, sem0, sem1):
        wid = lax.axis_index("s") * _NC + lax.axis_index("c")
        base = wid * rows_per_w
        xbufs = (xb0, xb1)
        bbufs = (bb0, bb1)
        sems = (sem0, sem1)

        lane = lax.iota(jnp.int32, _L)
        zero_i = jnp.zeros((_L,), jnp.int32)
        one_i = jnp.full((_L,), 1, jnp.int32)
        ones_f = jnp.ones((_L,), jnp.float32)
        lane0 = lane == 0
        perms = [lane ^ 8, lane ^ 4, lane ^ 2, lane ^ 1]
        cols = [jnp.full((_L,), k * _L, jnp.int32) + lane for k in range(n_win)]
        if tail:
            cols.append(jnp.full((_L,), seq_len - _L, jnp.int32) + lane)
            tail_new = lane >= (_L - tail)

        def start(c):
            r0 = base + c * _CH
            hx = pltpu.async_copy(
                x_hbm.at[pl.ds(r0, _CH), :], xbufs[c % 2], sems[c % 2])
            hb = pltpu.async_copy(
                bid_hbm.at[pl.ds(r0, _CH), :], bbufs[c % 2], sems[c % 2])
            return hx, hb

        handles = start(0)
        for c in range(n_ch):
            nxt = start(c + 1) if c + 1 < n_ch else None
            handles[0].wait()
            handles[1].wait()
            xv = xbufs[c % 2]
            bv = bbufs[c % 2]

            def grp(g, carry):
                rows16 = g * _L + lane
                mp_vec = plsc.load_gather(bv, [rows16, zero_i])
                bid_vec = plsc.load_gather(bv, [rows16, one_i])
                xmp_vec = plsc.load_gather(xv, [rows16, mp_vec])

                def row(jr, cc):
                    r = g * _L + jr
                    rsp = jnp.full((_L,), jr, jnp.int32)
                    bid_r = jnp.take(bid_vec, rsp, axis=0)
                    mp_r = jnp.take(mp_vec, rsp, axis=0)

                    acc = [ones_f, ones_f, ones_f, ones_f]
                    for k in range(n_win):
                        v = xv[r, pl.ds(k * _L, _L)]
                        ck = cols[k]
                        e = k & 1
                        acc[e] = jnp.where(ck < bid_r, acc[e] * v, acc[e])
                        acc[2 + e] = jnp.where(
                            ck < mp_r, acc[2 + e] * v, acc[2 + e])
                    if tail:
                        v = xv[r, pl.ds(seq_len - _L, _L)]
                        ck = cols[n_win]
                        e = n_win & 1
                        m_s = tail_new & (ck < bid_r)
                        m_2 = tail_new & (ck < mp_r)
                        acc[e] = jnp.where(m_s, acc[e] * v, acc[e])
                        acc[2 + e] = jnp.where(m_2, acc[2 + e] * v, acc[2 + e])
                    acc_s = acc[0] * acc[1]
                    acc_2 = acc[2] * acc[3]
                    for p in perms:
                        acc_s = acc_s * jnp.take(acc_s, p, axis=0)
                        acc_2 = acc_2 * jnp.take(acc_2, p, axis=0)
                    out0 = c * _CH + r
                    plsc.store_compressed(
                        sv.at[pl.ds(out0, _L)], acc_s, mask=lane0)
                    plsc.store_compressed(
                        rv.at[pl.ds(out0, _L)], acc_2, mask=lane0)
                    return cc

                lax.fori_loop(0, _L, row, 0, unroll=4)
                gout = c * _CH + g * _L
                p2 = rv[pl.ds(gout, _L)]
                rate = jnp.where(
                    mp_vec != zero_i, p2 * (1.0 - xmp_vec), jnp.float32(_EPS))
                rv[pl.ds(gout, _L)] = rate
                return carry

            lax.fori_loop(0, grp_per_ch, grp, 0)
            handles = nxt

        pltpu.sync_copy(sv.at[pl.ds(0, rows_per_w)],
                        surv_hbm.at[pl.ds(base, rows_per_w)])
        pltpu.sync_copy(rv.at[pl.ds(0, rows_per_w)],
                        rate_hbm.at[pl.ds(base, rows_per_w)])

    return sc_kernel


def kernel(bid_info, x):
    n, seq_len = x.shape
    surv, rate = _build(n, seq_len)(bid_info, x)
    return surv[:, None], rate[:, None]
